# baseline (speedup 1.0000x reference)
import functools

import jax
import jax.numpy as jnp
from jax.experimental import pallas as pl
from jax.experimental.pallas import tpu as pltpu

F32 = jnp.float32
_HIGH = jax.lax.Precision.HIGHEST
_HPREC = jax.lax.Precision.DEFAULT
_C1PREC = jax.lax.Precision.DEFAULT
_CPREC = jax.lax.Precision.DEFAULT
_CONVMODE = "im2col"


def _conv_body(x_ref, w_ref, cb_ref, s_ref, bb_ref, m_ref, v_ref, o_ref, *,
               H, W, Cin, Cout, NB, pool, mean, prec):
    scale = s_ref[...] * jax.lax.rsqrt(v_ref[...] + 1e-5)
    beta = (cb_ref[...] - m_ref[...]) * scale + bb_ref[...]
    M = NB * H * W
    if _CONVMODE == "im2col":
        patches = jnp.concatenate(
            [x_ref[:, dh:dh + H, dw:dw + W, :].reshape(M, Cin)
             for dh in range(3) for dw in range(3)], axis=1)
        acc = jnp.dot(patches, w_ref[...].reshape(9 * Cin, Cout),
                      preferred_element_type=F32, precision=prec)
    else:
        acc = jnp.zeros((M, Cout), F32)
        for dh in range(3):
            for dw in range(3):
                xs = x_ref[:, dh:dh + H, dw:dw + W, :].reshape(M, Cin)
                acc = acc + jnp.dot(xs, w_ref[dh * 3 + dw],
                                    preferred_element_type=F32, precision=prec)
    y = jnp.maximum(acc * scale + beta, 0.0)
    if pool:
        y = y.reshape(NB, H // 2, 2, W // 2, 2, Cout)
        y = jnp.max(y, axis=(2, 4))
        if mean:
            o_ref[...] = jnp.mean(y, axis=(1, 2))
        else:
            o_ref[...] = y
    else:
        o_ref[...] = y.reshape(NB, H, W, Cout)


def _conv_layer(h, w, cb, bns, bnb, bnm, bnv, *, pool, mean, NB,
                prec=jax.lax.Precision.DEFAULT):
    B, H, W, Cin = h.shape
    Cout = w.shape[0]
    hp = jnp.pad(h, ((0, 0), (1, 1), (1, 1), (0, 0)))
    wt = jnp.transpose(w, (2, 3, 1, 0)).reshape(9, Cin, Cout)
    vecs = [v.reshape(1, Cout) for v in (cb, bns, bnb, bnm, bnv)]
    if mean:
        out_shape = jax.ShapeDtypeStruct((B, Cout), F32)
        o_spec = pl.BlockSpec((NB, Cout), lambda i: (i, 0))
    elif pool:
        out_shape = jax.ShapeDtypeStruct((B, H // 2, W // 2, Cout), F32)
        o_spec = pl.BlockSpec((NB, H // 2, W // 2, Cout), lambda i: (i, 0, 0, 0))
    else:
        out_shape = jax.ShapeDtypeStruct((B, H, W, Cout), F32)
        o_spec = pl.BlockSpec((NB, H, W, Cout), lambda i: (i, 0, 0, 0))
    body = functools.partial(_conv_body, H=H, W=W, Cin=Cin, Cout=Cout,
                             NB=NB, pool=pool, mean=mean, prec=prec)
    return pl.pallas_call(
        body,
        grid=(B // NB,),
        in_specs=[
            pl.BlockSpec((NB, H + 2, W + 2, Cin), lambda i: (i, 0, 0, 0)),
            pl.BlockSpec((9, Cin, Cout), lambda i: (0, 0, 0)),
        ] + [pl.BlockSpec((1, Cout), lambda i: (0, 0))] * 5,
        out_specs=o_spec,
        out_shape=out_shape,
    )(hp, wt, *vecs)


def _csum(v):
    n, B = v.shape
    m = ((n + 7) // 8) * 8
    if m != n:
        v = jnp.concatenate([v, jnp.zeros((m - n, B), v.dtype)], axis=0)
    acc = v[0:8, :]
    for i in range(1, m // 8):
        acc = acc + v[8 * i:8 * i + 8, :]
    a = acc[0:4, :] + acc[4:8, :]
    a = a[0:2, :] + a[2:4, :]
    return a[0:1, :] + a[1:2, :]


def _csum_seq(v):
    acc = v[0:1, :]
    for i in range(1, v.shape[0]):
        acc = acc + v[i:i + 1, :]
    return acc


_CS_S = _csum
_CS_E = _csum


def _head_body(f_ref, w_ref, b_ref, fin_ref, conf_ref, d_ref, lg_ref):
    B = f_ref.shape[0]
    E, C, _ = w_ref.shape
    feats_t = jnp.transpose(f_ref[...])
    rows = []
    for e in range(E):
        lg_t = jnp.dot(w_ref[e], feats_t, preferred_element_type=F32,
                       precision=_HPREC) + b_ref[e]
        lg_ref[e] = lg_t
        mx = jnp.max(lg_t, axis=0, keepdims=True)
        ex = jnp.exp(lg_t - mx)
        p = ex / _CS_S(ex)
        ent = -_CS_E(p * jnp.log(jnp.clip(p, 1e-12, None)))
        rows.append(-ent)
    conf_t = jnp.concatenate(rows, axis=0)
    conf = jnp.transpose(conf_t)
    conf_ref[...] = conf
    row = jax.lax.broadcasted_iota(jnp.int32, (B, B), 0)
    col = jax.lax.broadcasted_iota(jnp.int32, (B, B), 1)
    ltm = col < row
    ceff = float(min(64, B))
    cap_cols = []
    for e in range(E):
        cb = conf[:, e:e + 1]
        rb = conf_t[e:e + 1, :]
        gt = (rb > cb).astype(F32)
        eq = ((rb == cb) & ltm).astype(F32)
        rank = jnp.sum(gt + eq, axis=1, keepdims=True)
        cap_cols.append((rank < ceff).astype(F32))
    incap = jnp.concatenate(cap_cols, axis=1)
    masked = jnp.where(incap > 0.5, conf, -1000000000.0)
    eio = jax.lax.broadcasted_iota(jnp.int32, (1, E), 1)
    d_cols = []
    for e in range(E):
        cb = masked[:, e:e + 1]
        gt = (masked > cb).astype(F32)
        eq = ((masked == cb) & (eio < e)).astype(F32)
        rank = jnp.sum(gt + eq, axis=1, keepdims=True)
        d_cols.append((rank < 2.0).astype(F32))
    d = jnp.concatenate(d_cols, axis=1)
    d_ref[...] = d
    wgt_t = jnp.transpose(conf * d)
    norm = jnp.clip(jnp.sum(d, axis=1, keepdims=True), 1.0, None)
    acc_t = jnp.zeros((C, B), F32)
    for e in range(E):
        acc_t = acc_t + wgt_t[e:e + 1, :] * lg_ref[e]
    fin_ref[...] = jnp.transpose(acc_t) / norm


def _moe_head(feats, cls_weight, cls_bias):
    B, Dm = feats.shape
    E, C, _ = cls_weight.shape
    bias = cls_bias.reshape(E, C, 1)
    out_shapes = (
        jax.ShapeDtypeStruct((B, C), F32),
        jax.ShapeDtypeStruct((B, E), F32),
        jax.ShapeDtypeStruct((B, E), F32),
    )
    final, conf, d = pl.pallas_call(
        _head_body,
        in_specs=[
            pl.BlockSpec((B, Dm), lambda: (0, 0)),
            pl.BlockSpec((E, C, Dm), lambda: (0, 0, 0)),
            pl.BlockSpec((E, C, 1), lambda: (0, 0, 0)),
        ],
        out_specs=(
            pl.BlockSpec((B, C), lambda: (0, 0)),
            pl.BlockSpec((B, E), lambda: (0, 0)),
            pl.BlockSpec((B, E), lambda: (0, 0)),
        ),
        out_shape=out_shapes,
        scratch_shapes=[pltpu.VMEM((E, C, B), F32)],
    )(feats, cls_weight, bias)
    return final, conf, d.astype(bool)


def kernel(x, conv1_w, conv1_b, bn1_s, bn1_b, bn1_m, bn1_v,
           conv2_w, conv2_b, bn2_s, bn2_b, bn2_m, bn2_v,
           conv3_w, conv3_b, bn3_s, bn3_b, bn3_m, bn3_v,
           conv4_w, conv4_b, bn4_s, bn4_b, bn4_m, bn4_v,
           cls_weight, cls_bias):
    h = jnp.transpose(x, (0, 2, 3, 1))
    h = _conv_layer(h, conv1_w, conv1_b, bn1_s, bn1_b, bn1_m, bn1_v,
                    pool=False, mean=False, NB=2, prec=_C1PREC)
    h = _conv_layer(h, conv2_w, conv2_b, bn2_s, bn2_b, bn2_m, bn2_v,
                    pool=True, mean=False, NB=2, prec=_CPREC)
    h = _conv_layer(h, conv3_w, conv3_b, bn3_s, bn3_b, bn3_m, bn3_v,
                    pool=False, mean=False, NB=8, prec=_CPREC)
    feats = _conv_layer(h, conv4_w, conv4_b, bn4_s, bn4_b, bn4_m, bn4_v,
                        pool=True, mean=True, NB=8, prec=_CPREC)
    return _moe_head(feats, cls_weight, cls_bias)
